# baseline (speedup 1.0000x reference)
import functools

import jax
import jax.numpy as jnp
from jax import lax
from jax.experimental import pallas as pl
from jax.experimental.pallas import tpu as pltpu
from jax.experimental.pallas import tpu_sc as plsc

B, T, D = 4, 2048, 768
VOCAB = 50257
NC, NS, L = 2, 16, 16
NW = NC * NS
TPW = T // NW
CS = 32
NCPB = TPW // CS
NCHUNK = B * NCPB
NBUF = 3
LEAD = NBUF - 1
NSPLIT = 2

_mesh = plsc.VectorSubcoreMesh(
    core_axis_name="c", subcore_axis_name="s", num_cores=NC, num_subcores=NS
)


@functools.partial(
    pl.kernel,
    out_type=jax.ShapeDtypeStruct((B, T, D), jnp.float32),
    mesh=_mesh,
    scratch_types=[
        pltpu.VMEM((NCHUNK, CS), jnp.int32),
        pltpu.VMEM((TPW, D), jnp.float32),
        [pltpu.VMEM((CS, D), jnp.float32) for _ in range(NBUF)],
        pltpu.SemaphoreType.DMA,
        pltpu.SemaphoreType.DMA,
        [pltpu.SemaphoreType.DMA for _ in range(NBUF)],
        [pltpu.SemaphoreType.DMA for _ in range(NBUF)],
    ],
)
def _emb_lookup(ids_hbm, wte_hbm, wpe_hbm, out_hbm,
                idx_v, wpe_v, rows_v, isem, wsem, gsems, ssems):
    wid = lax.axis_index("s") * NC + lax.axis_index("c")
    t0 = wid * TPW

    idx_descs = []
    for k in range(NCHUNK):
        b, c = k // NCPB, k % NCPB
        idx_descs.append(
            pltpu.async_copy(
                ids_hbm.at[b, pl.ds(t0 + c * CS, CS)], idx_v.at[k], isem
            )
        )
    wpe_desc = pltpu.async_copy(wpe_hbm.at[pl.ds(t0, TPW), :], wpe_v, wsem)
    for d in idx_descs:
        d.wait()

    HCS = CS // NSPLIT

    def start_gather(k, p):
        return pltpu.async_copy(wte_hbm.at[idx_v.at[k]], rows_v[p], gsems[p])

    def start_store_half(k, p, h):
        b, c = k // NCPB, k % NCPB
        return pltpu.async_copy(
            rows_v[p].at[pl.ds(h * HCS, HCS)],
            out_hbm.at[b, pl.ds(t0 + c * CS + h * HCS, HCS), :],
            ssems[p],
        )

    def add_wpe_half(k, p, h):
        c = k % NCPB
        rp = rows_v[p]

        @plsc.parallel_loop(h * HCS, (h + 1) * HCS)
        def _(i):
            for j in range(D // L):
                sl = pl.ds(j * L, L)
                plsc.addupdate(rp.at[i, sl], wpe_v[c * CS + i, sl])

    g_descs = {}
    s_descs = {}
    for k in range(NBUF):
        g_descs[k] = start_gather(k, k)
    wpe_desc.wait()
    for k in range(NCHUNK):
        p = k % NBUF
        g_descs[k].wait()
        s_descs[k] = []
        for h in range(NSPLIT):
            add_wpe_half(k, p, h)
            s_descs[k].append(start_store_half(k, p, h))
        j = k + LEAD
        jw = j - NBUF
        if j < NCHUNK and jw >= 0:
            for d in s_descs[jw]:
                d.wait()
            g_descs[j] = start_gather(j, j % NBUF)
    for k in range(max(0, NCHUNK - NBUF), NCHUNK):
        for d in s_descs[k]:
            d.wait()


def kernel(input_ids, wte, wpe):
    ids32 = input_ids.astype(jnp.int32)
    return _emb_lookup(ids32, wte, wpe)


# speedup vs baseline: 1.0691x; 1.0691x over previous
import functools

import jax
import jax.numpy as jnp
from jax import lax
from jax.experimental import pallas as pl
from jax.experimental.pallas import tpu as pltpu
from jax.experimental.pallas import tpu_sc as plsc

B, T, D = 4, 2048, 768
VOCAB = 50257
NC, NS, L = 2, 16, 16
NW = NC * NS
TPW = T // NW
CS = 8
NGRP = TPW // CS
NBUF = 3

_mesh = plsc.VectorSubcoreMesh(
    core_axis_name="c", subcore_axis_name="s", num_cores=NC, num_subcores=NS
)


@functools.partial(
    pl.kernel,
    out_type=jax.ShapeDtypeStruct((B, T, D), jnp.float32),
    mesh=_mesh,
    scratch_types=[
        pltpu.VMEM((B, TPW), jnp.int32),
        pltpu.VMEM((TPW, D), jnp.float32),
        [[pltpu.VMEM((CS, D), jnp.float32) for _ in range(B)]
         for _ in range(NBUF)],
        pltpu.SemaphoreType.DMA,
        pltpu.SemaphoreType.DMA,
        [pltpu.SemaphoreType.DMA for _ in range(NBUF)],
        [pltpu.SemaphoreType.DMA for _ in range(NBUF)],
    ],
)
def _emb_lookup(ids_hbm, wte_hbm, wpe_hbm, out_hbm,
                idx_v, wpe_v, rows_v, isem, wsem, gsems, ssems):
    sid = lax.axis_index("s")
    wid = sid * NC + lax.axis_index("c")
    t0 = wid * TPW

    idx_descs = [
        pltpu.async_copy(
            ids_hbm.at[b, pl.ds(t0, TPW)], idx_v.at[b], isem
        )
        for b in range(B)
    ]
    wpe_desc = pltpu.async_copy(wpe_hbm.at[pl.ds(t0, TPW), :], wpe_v, wsem)
    for d in idx_descs:
        d.wait()

    def start_gathers(g):
        p = g % NBUF
        return [
            pltpu.async_copy(
                wte_hbm.at[idx_v.at[b, pl.ds(g * CS, CS)]],
                rows_v[p][b],
                gsems[p],
            )
            for b in range(B)
        ]

    def start_stores(g):
        p = g % NBUF
        return [
            pltpu.async_copy(
                rows_v[p][b],
                out_hbm.at[b, pl.ds(t0 + g * CS, CS), :],
                ssems[p],
            )
            for b in range(B)
        ]

    def add_wpe(g):
        p = g % NBUF
        bufs = rows_v[p]

        @plsc.parallel_loop(0, CS, unroll=1)
        def _(i):
            for j in range(D // L):
                sl = pl.ds(j * L, L)
                x = wpe_v[g * CS + i, sl]
                for b in range(B):
                    plsc.addupdate(bufs[b].at[i, sl], x)

    g_descs = {}
    s_descs = {}
    for g in range(NBUF):
        g_descs[g] = start_gathers(g)
    wpe_desc.wait()
    for g in range(NGRP):
        for d in g_descs[g]:
            d.wait()
        add_wpe(g)
        s_descs[g] = start_stores(g)
        j = g + NBUF - 1
        if 1 <= g and j < NGRP:
            for d in s_descs[g - 1]:
                d.wait()
            g_descs[j] = start_gathers(j)
    for g in range(max(0, NGRP - NBUF), NGRP):
        for d in s_descs[g]:
            d.wait()


def kernel(input_ids, wte, wpe):
    ids32 = input_ids.astype(jnp.int32)
    return _emb_lookup(ids32, wte, wpe)


# speedup vs baseline: 1.0711x; 1.0019x over previous
import functools

import jax
import jax.numpy as jnp
from jax import lax
from jax.experimental import pallas as pl
from jax.experimental.pallas import tpu as pltpu
from jax.experimental.pallas import tpu_sc as plsc

B, T, D = 4, 2048, 768
VOCAB = 50257
NC, NS, L = 2, 16, 16
NW = NC * NS
TPW = T // NW
CS = 8
NGRP = TPW // CS
NBUF = 3

_mesh = plsc.VectorSubcoreMesh(
    core_axis_name="c", subcore_axis_name="s", num_cores=NC, num_subcores=NS
)


@functools.partial(
    pl.kernel,
    out_type=jax.ShapeDtypeStruct((B, T, D), jnp.float32),
    mesh=_mesh,
    scratch_types=[
        pltpu.VMEM((B, TPW), jnp.int32),
        pltpu.VMEM((TPW, D), jnp.float32),
        [[pltpu.VMEM((CS, D), jnp.float32) for _ in range(B)]
         for _ in range(NBUF)],
        pltpu.SemaphoreType.DMA,
        pltpu.SemaphoreType.DMA,
        [pltpu.SemaphoreType.DMA for _ in range(NBUF)],
        [pltpu.SemaphoreType.DMA for _ in range(NBUF)],
    ],
)
def _emb_lookup(ids_hbm, wte_hbm, wpe_hbm, out_hbm,
                idx_v, wpe_v, rows_v, isem, wsem, gsems, ssems):
    sid = lax.axis_index("s")
    wid = sid * NC + lax.axis_index("c")
    t0 = wid * TPW

    idx_descs = [
        pltpu.async_copy(
            ids_hbm.at[b, pl.ds(t0, TPW)], idx_v.at[b], isem
        )
        for b in range(B)
    ]
    wpe_desc = pltpu.async_copy(wpe_hbm.at[pl.ds(t0, TPW), :], wpe_v, wsem)
    for d in idx_descs:
        d.wait()

    def start_gathers(g):
        p = g % NBUF
        return [
            pltpu.async_copy(
                wte_hbm.at[idx_v.at[b, pl.ds(g * CS, CS)]],
                rows_v[p][b],
                gsems[p],
            )
            for b in range(B)
        ]

    def start_stores(g):
        p = g % NBUF
        return [
            pltpu.async_copy(
                rows_v[p][b],
                out_hbm.at[b, pl.ds(t0 + g * CS, CS), :],
                ssems[p],
            )
            for b in range(B)
        ]

    def add_wpe(g):
        p = g % NBUF
        bufs = rows_v[p]

        @plsc.parallel_loop(0, CS, unroll=1)
        def _(i):
            for j in range(D // L):
                sl = pl.ds(j * L, L)
                x = wpe_v[g * CS + i, sl]
                for b in range(B):
                    plsc.addupdate(bufs[b].at[i, sl], x)

    g_descs = {}
    s_descs = {}
    for g in range(NBUF):
        g_descs[g] = start_gathers(g)
    wpe_desc.wait()
    for g in range(NGRP):
        for d in g_descs[g]:
            d.wait()
        j = g + NBUF - 1
        if 1 <= g and j < NGRP:
            for d in s_descs[g - 1]:
                d.wait()
            g_descs[j] = start_gathers(j)
        add_wpe(g)
        s_descs[g] = start_stores(g)
    for g in range(max(0, NGRP - NBUF), NGRP):
        for d in s_descs[g]:
            d.wait()


def kernel(input_ids, wte, wpe):
    ids32 = input_ids.astype(jnp.int32)
    return _emb_lookup(ids32, wte, wpe)


# speedup vs baseline: 1.0849x; 1.0129x over previous
import functools

import jax
import jax.numpy as jnp
from jax import lax
from jax.experimental import pallas as pl
from jax.experimental.pallas import tpu as pltpu
from jax.experimental.pallas import tpu_sc as plsc

B, T, D = 4, 2048, 768
VOCAB = 50257
NC, NS, L = 2, 16, 16
NW = NC * NS
TPW = T // NW
CS = 16
NGRP = TPW // CS
NBUF = 2

_mesh = plsc.VectorSubcoreMesh(
    core_axis_name="c", subcore_axis_name="s", num_cores=NC, num_subcores=NS
)


@functools.partial(
    pl.kernel,
    out_type=jax.ShapeDtypeStruct((B, T, D), jnp.float32),
    mesh=_mesh,
    scratch_types=[
        pltpu.VMEM((B, TPW), jnp.int32),
        [pltpu.VMEM((CS, D), jnp.float32) for _ in range(NBUF)],
        [[pltpu.VMEM((CS, D), jnp.float32) for _ in range(B)]
         for _ in range(NBUF)],
        pltpu.SemaphoreType.DMA,
        [pltpu.SemaphoreType.DMA for _ in range(NBUF)],
        [pltpu.SemaphoreType.DMA for _ in range(NBUF)],
        [pltpu.SemaphoreType.DMA for _ in range(NBUF)],
    ],
)
def _emb_lookup(ids_hbm, wte_hbm, wpe_hbm, out_hbm,
                idx_v, wpe_v, rows_v, isem, wsems, gsems, ssems):
    sid = lax.axis_index("s")
    wid = sid * NC + lax.axis_index("c")
    t0 = wid * TPW

    idx_descs = [
        pltpu.async_copy(
            ids_hbm.at[b, pl.ds(t0, TPW)], idx_v.at[b], isem
        )
        for b in range(B)
    ]
    for d in idx_descs:
        d.wait()

    def start_gathers(g):
        p = g % NBUF
        return [
            pltpu.async_copy(
                wte_hbm.at[idx_v.at[b, pl.ds(g * CS, CS)]],
                rows_v[p][b],
                gsems[p],
            )
            for b in range(B)
        ]

    def start_wpe(g):
        p = g % NBUF
        return pltpu.async_copy(
            wpe_hbm.at[pl.ds(t0 + g * CS, CS), :], wpe_v[p], wsems[p]
        )

    def start_stores(g):
        p = g % NBUF
        return [
            pltpu.async_copy(
                rows_v[p][b],
                out_hbm.at[b, pl.ds(t0 + g * CS, CS), :],
                ssems[p],
            )
            for b in range(B)
        ]

    def add_wpe(g):
        p = g % NBUF
        bufs = rows_v[p]
        wp = wpe_v[p]

        @plsc.parallel_loop(0, CS, unroll=1)
        def _(i):
            for j in range(D // L):
                sl = pl.ds(j * L, L)
                x = wp[i, sl]
                for b in range(B):
                    plsc.addupdate(bufs[b].at[i, sl], x)

    g_descs = {}
    w_descs = {}
    s_descs = {}
    for g in range(NBUF):
        g_descs[g] = start_gathers(g)
        w_descs[g] = start_wpe(g)
    for g in range(NGRP):
        for d in g_descs[g]:
            d.wait()
        w_descs[g].wait()
        j = g + NBUF - 1
        if 1 <= g and j < NGRP:
            for d in s_descs[g - 1]:
                d.wait()
            g_descs[j] = start_gathers(j)
            w_descs[j] = start_wpe(j)
        add_wpe(g)
        s_descs[g] = start_stores(g)
    for g in range(max(0, NGRP - NBUF), NGRP):
        for d in s_descs[g]:
            d.wait()


def kernel(input_ids, wte, wpe):
    ids32 = input_ids.astype(jnp.int32)
    return _emb_lookup(ids32, wte, wpe)
